# Initial kernel scaffold; baseline (speedup 1.0000x reference)
import jax
import jax.numpy as jnp
from jax.experimental import pallas as pl


def kernel(xyz, new_xyz, features):
    raise NotImplementedError("write your pallas kernel here")


# speedup vs baseline: 21.0261x; 21.0261x over previous
import functools

import jax
import jax.numpy as jnp
from jax import lax
from jax.experimental import pallas as pl
from jax.experimental.pallas import tpu as pltpu
from jax.experimental.pallas import tpu_sc as plsc

RADIUS2 = 0.2 * 0.2
NSMP = 32
L = 16
NW = 32


def _wid():
    return lax.axis_index("s") * 2 + lax.axis_index("c")


@functools.lru_cache(maxsize=None)
def _make_k1(B, N, S):
    NB = NW // B
    SQ = S // NB
    NCH = N // L
    mesh = plsc.VectorSubcoreMesh(core_axis_name="c", subcore_axis_name="s")

    @functools.partial(
        pl.kernel,
        out_type=[
            jax.ShapeDtypeStruct((B * S * NSMP,), jnp.int32),
            jax.ShapeDtypeStruct((B * 3 * S * NSMP,), jnp.float32),
        ],
        mesh=mesh,
        compiler_params=pltpu.CompilerParams(needs_layout_passes=False),
        scratch_types=[
            pltpu.VMEM((3 * N,), jnp.float32),
            pltpu.VMEM((3 * SQ,), jnp.float32),
            pltpu.VMEM((64,), jnp.int32),
            pltpu.VMEM((SQ * NSMP,), jnp.int32),
            pltpu.VMEM((3 * SQ * NSMP,), jnp.float32),
        ],
    )
    def k1(xyz_hbm, q_hbm, idx_hbm, gx_hbm, pts, qrs, idxbuf, idx_out, gx_out):
        wid = _wid()
        b = wid // NB
        q0 = (wid % NB) * SQ
        for c in range(3):
            pltpu.sync_copy(xyz_hbm.at[pl.ds((b * 3 + c) * N, N)],
                            pts.at[pl.ds(c * N, N)])
            pltpu.sync_copy(q_hbm.at[pl.ds((b * 3 + c) * S + q0, SQ)],
                            qrs.at[pl.ds(c * SQ, SQ)])
        iota = lax.broadcasted_iota(jnp.int32, (L,), 0)
        zeros16 = jnp.zeros((L,), jnp.int32)

        def per_query(s, carry):
            idxbuf[pl.ds(0, L)] = zeros16
            qx = plsc.load_gather(qrs, [jnp.full((L,), s, jnp.int32)])
            qy = plsc.load_gather(qrs, [jnp.full((L,), SQ + s, jnp.int32)])
            qz = plsc.load_gather(qrs, [jnp.full((L,), 2 * SQ + s, jnp.int32)])

            def cond(cr):
                ch, cnt = cr
                return jnp.logical_and(cnt < NSMP, ch < NCH)

            def body(cr):
                ch, cnt = cr
                i0 = ch * L
                vx = pts[pl.ds(i0, L)]
                vy = pts[pl.ds(N + i0, L)]
                vz = pts[pl.ds(2 * N + i0, L)]
                dx = vx - qx
                dy = vy - qy
                dz = vz - qz
                d2 = dx * dx + dy * dy + dz * dz
                m = d2 < RADIUS2
                plsc.store_compressed(idxbuf.at[pl.ds(cnt, L)], iota + i0, mask=m)
                npts = jnp.max(plsc.all_reduce_population_count(m))
                return ch + 1, cnt + npts

            _, cnt = lax.while_loop(cond, body, (jnp.int32(0), jnp.int32(0)))

            first = plsc.load_gather(idxbuf, [zeros16])
            sel0 = jnp.where(iota < cnt, idxbuf[pl.ds(0, L)], first)
            sel1 = jnp.where(iota + L < cnt, idxbuf[pl.ds(L, L)], first)
            idx_out[pl.ds(s * NSMP, L)] = sel0
            idx_out[pl.ds(s * NSMP + L, L)] = sel1
            for ci, qv in enumerate((qx, qy, qz)):
                g0 = plsc.load_gather(pts, [sel0 + ci * N]) - qv
                g1 = plsc.load_gather(pts, [sel1 + ci * N]) - qv
                gx_out[pl.ds((ci * SQ + s) * NSMP, L)] = g0
                gx_out[pl.ds((ci * SQ + s) * NSMP + L, L)] = g1
            return carry

        lax.fori_loop(0, SQ, per_query, 0)
        pltpu.sync_copy(idx_out,
                        idx_hbm.at[pl.ds((b * S + q0) * NSMP, SQ * NSMP)])
        for c in range(3):
            pltpu.sync_copy(
                gx_out.at[pl.ds(c * SQ * NSMP, SQ * NSMP)],
                gx_hbm.at[pl.ds(((b * 3 + c) * S + q0) * NSMP, SQ * NSMP)])

    return k1


@functools.lru_cache(maxsize=None)
def _make_k2(B, N, S, C):
    CPT = (B * C) // NW
    CH = 128
    NQC = S // CH
    KPC = CH * NSMP // L
    mesh = plsc.VectorSubcoreMesh(core_axis_name="c", subcore_axis_name="s")

    @functools.partial(
        pl.kernel,
        out_type=jax.ShapeDtypeStruct((B * C * S * NSMP,), jnp.float32),
        mesh=mesh,
        compiler_params=pltpu.CompilerParams(needs_layout_passes=False),
        scratch_types=[
            pltpu.VMEM((CPT * N,), jnp.float32),
            pltpu.VMEM((CH * NSMP,), jnp.int32),
            pltpu.VMEM((CPT * CH * NSMP,), jnp.float32),
        ],
    )
    def k2(feat_hbm, idx_hbm, out_hbm, feats, idxc, outb):
        wid = _wid()
        u0 = wid * CPT
        b = u0 // C
        for j in range(CPT):
            pltpu.sync_copy(feat_hbm.at[pl.ds((u0 + j) * N, N)],
                            feats.at[pl.ds(j * N, N)])

        def per_chunk(qc, carry):
            pltpu.sync_copy(
                idx_hbm.at[pl.ds((b * S + qc * CH) * NSMP, CH * NSMP)], idxc)

            def per_vreg(k, c2):
                ids = idxc[pl.ds(k * L, L)]
                for j in range(CPT):
                    outb[pl.ds(j * CH * NSMP + k * L, L)] = plsc.load_gather(
                        feats.at[pl.ds(j * N, N)], [ids])
                return c2

            lax.fori_loop(0, KPC, per_vreg, 0)
            for j in range(CPT):
                pltpu.sync_copy(
                    outb.at[pl.ds(j * CH * NSMP, CH * NSMP)],
                    out_hbm.at[pl.ds(((u0 + j) * S + qc * CH) * NSMP,
                                     CH * NSMP)])
            return carry

        lax.fori_loop(0, NQC, per_chunk, 0)

    return k2


def kernel(xyz, new_xyz, features):
    B, N, _ = xyz.shape
    S = new_xyz.shape[1]
    C = features.shape[1]
    xyz_t = jnp.transpose(xyz, (0, 2, 1)).reshape(-1)
    q_t = jnp.transpose(new_xyz, (0, 2, 1)).reshape(-1)
    idx_flat, gx_flat = _make_k1(B, N, S)(xyz_t, q_t)
    gf_flat = _make_k2(B, N, S, C)(features.reshape(-1), idx_flat)
    gx = gx_flat.reshape(B, 3, S, NSMP)
    gf = gf_flat.reshape(B, C, S, NSMP)
    return jnp.concatenate([gx, gf], axis=1)


# speedup vs baseline: 26.6393x; 1.2670x over previous
import functools

import jax
import jax.numpy as jnp
from jax import lax
from jax.experimental import pallas as pl
from jax.experimental.pallas import tpu as pltpu
from jax.experimental.pallas import tpu_sc as plsc

RADIUS2 = 0.2 * 0.2
NSMP = 32
L = 16
NW = 32


def _wid():
    return lax.axis_index("s") * 2 + lax.axis_index("c")


@functools.lru_cache(maxsize=None)
def _make_k1(B, N, S):
    NB = NW // B
    SQ = S // NB
    NCH = N // L
    mesh = plsc.VectorSubcoreMesh(core_axis_name="c", subcore_axis_name="s")

    @functools.partial(
        pl.kernel,
        out_type=[
            jax.ShapeDtypeStruct((B * S * NSMP,), jnp.int32),
            jax.ShapeDtypeStruct((B * 3 * S * NSMP,), jnp.float32),
        ],
        mesh=mesh,
        compiler_params=pltpu.CompilerParams(needs_layout_passes=False),
        scratch_types=[
            pltpu.VMEM((3 * N,), jnp.float32),
            pltpu.VMEM((3 * SQ,), jnp.float32),
            pltpu.VMEM((64,), jnp.int32),
            pltpu.VMEM((SQ * NSMP,), jnp.int32),
            pltpu.VMEM((3 * SQ * NSMP,), jnp.float32),
        ],
    )
    def k1(xyz_hbm, q_hbm, idx_hbm, gx_hbm, pts, qrs, idxbuf, idx_out, gx_out):
        wid = _wid()
        b = wid // NB
        q0 = (wid % NB) * SQ
        for c in range(3):
            pltpu.sync_copy(xyz_hbm.at[pl.ds((b * 3 + c) * N, N)],
                            pts.at[pl.ds(c * N, N)])
            pltpu.sync_copy(q_hbm.at[pl.ds((b * 3 + c) * S + q0, SQ)],
                            qrs.at[pl.ds(c * SQ, SQ)])
        iota = lax.broadcasted_iota(jnp.int32, (L,), 0)
        zeros16 = jnp.zeros((L,), jnp.int32)

        def per_query(s, carry):
            idxbuf[pl.ds(0, L)] = zeros16
            qx = plsc.load_gather(qrs, [jnp.full((L,), s, jnp.int32)])
            qy = plsc.load_gather(qrs, [jnp.full((L,), SQ + s, jnp.int32)])
            qz = plsc.load_gather(qrs, [jnp.full((L,), 2 * SQ + s, jnp.int32)])

            def cond(cr):
                ch, cnt = cr
                return jnp.logical_and(cnt < NSMP, ch < NCH // 2)

            def body(cr):
                ch, cnt = cr
                i0 = ch * (2 * L)
                cnt1 = cnt
                for h in range(2):
                    ib = i0 + h * L
                    vx = pts[pl.ds(ib, L)]
                    vy = pts[pl.ds(N + ib, L)]
                    vz = pts[pl.ds(2 * N + ib, L)]
                    dx = vx - qx
                    dy = vy - qy
                    dz = vz - qz
                    d2 = dx * dx + dy * dy + dz * dz
                    m = d2 < RADIUS2
                    plsc.store_compressed(idxbuf.at[pl.ds(cnt1, L)],
                                          iota + ib, mask=m)
                    cnt1 = cnt1 + plsc.all_reduce_population_count(m)[0]
                return ch + 1, cnt1

            _, cnt = lax.while_loop(cond, body, (jnp.int32(0), jnp.int32(0)))

            first = plsc.load_gather(idxbuf, [zeros16])
            sel0 = jnp.where(iota < cnt, idxbuf[pl.ds(0, L)], first)
            sel1 = jnp.where(iota + L < cnt, idxbuf[pl.ds(L, L)], first)
            idx_out[pl.ds(s * NSMP, L)] = sel0
            idx_out[pl.ds(s * NSMP + L, L)] = sel1
            for ci, qv in enumerate((qx, qy, qz)):
                g0 = plsc.load_gather(pts, [sel0 + ci * N]) - qv
                g1 = plsc.load_gather(pts, [sel1 + ci * N]) - qv
                gx_out[pl.ds((ci * SQ + s) * NSMP, L)] = g0
                gx_out[pl.ds((ci * SQ + s) * NSMP + L, L)] = g1
            return carry

        lax.fori_loop(0, SQ, per_query, 0)
        pltpu.sync_copy(idx_out,
                        idx_hbm.at[pl.ds((b * S + q0) * NSMP, SQ * NSMP)])
        for c in range(3):
            pltpu.sync_copy(
                gx_out.at[pl.ds(c * SQ * NSMP, SQ * NSMP)],
                gx_hbm.at[pl.ds(((b * 3 + c) * S + q0) * NSMP, SQ * NSMP)])

    return k1


@functools.lru_cache(maxsize=None)
def _make_k2(B, N, S, C):
    CPT = (B * C) // NW
    CH = 128
    NQC = S // CH
    KPC = CH * NSMP // L
    mesh = plsc.VectorSubcoreMesh(core_axis_name="c", subcore_axis_name="s")

    @functools.partial(
        pl.kernel,
        out_type=jax.ShapeDtypeStruct((B * C * S * NSMP,), jnp.float32),
        mesh=mesh,
        compiler_params=pltpu.CompilerParams(needs_layout_passes=False),
        scratch_types=[
            pltpu.VMEM((CPT * N,), jnp.float32),
            pltpu.VMEM((CH * NSMP,), jnp.int32),
            pltpu.VMEM((CPT * CH * NSMP,), jnp.float32),
        ],
    )
    def k2(feat_hbm, idx_hbm, out_hbm, feats, idxc, outb):
        wid = _wid()
        u0 = wid * CPT
        b = u0 // C
        for j in range(CPT):
            pltpu.sync_copy(feat_hbm.at[pl.ds((u0 + j) * N, N)],
                            feats.at[pl.ds(j * N, N)])

        def per_chunk(qc, carry):
            pltpu.sync_copy(
                idx_hbm.at[pl.ds((b * S + qc * CH) * NSMP, CH * NSMP)], idxc)

            def per_vreg(k, c2):
                ids = idxc[pl.ds(k * L, L)]
                for j in range(CPT):
                    outb[pl.ds(j * CH * NSMP + k * L, L)] = plsc.load_gather(
                        feats.at[pl.ds(j * N, N)], [ids])
                return c2

            lax.fori_loop(0, KPC, per_vreg, 0)
            for j in range(CPT):
                pltpu.sync_copy(
                    outb.at[pl.ds(j * CH * NSMP, CH * NSMP)],
                    out_hbm.at[pl.ds(((u0 + j) * S + qc * CH) * NSMP,
                                     CH * NSMP)])
            return carry

        lax.fori_loop(0, NQC, per_chunk, 0)

    return k2


def kernel(xyz, new_xyz, features):
    B, N, _ = xyz.shape
    S = new_xyz.shape[1]
    C = features.shape[1]
    xyz_t = jnp.transpose(xyz, (0, 2, 1)).reshape(-1)
    q_t = jnp.transpose(new_xyz, (0, 2, 1)).reshape(-1)
    idx_flat, gx_flat = _make_k1(B, N, S)(xyz_t, q_t)
    gf_flat = _make_k2(B, N, S, C)(features.reshape(-1), idx_flat)
    gx = gx_flat.reshape(B, 3, S, NSMP)
    gf = gf_flat.reshape(B, C, S, NSMP)
    return jnp.concatenate([gx, gf], axis=1)
